# baseline (speedup 1.0000x reference)
import functools

import jax
import jax.numpy as jnp
from jax import lax
from jax.experimental import pallas as pl
from jax.experimental.pallas import tpu as pltpu

EPS = 0.001
NS = 601
SMIN = -300.0

B, K = 4096, 50
CB = 32


def _tent_block(xt_ref, out_ref, tt_scr, *, cb):
    j = pl.program_id(0)

    @pl.when(j == 0)
    def _():
        x = xt_ref[...]
        tt_scr[...] = jnp.sign(x) * (
            jnp.sqrt(jnp.abs(x) + 1.0) - 1.0 + EPS * x)

    tt = tt_scr[...]
    col = lax.broadcasted_iota(jnp.int32, (K, cb, B), 1) + j * cb
    sup = col.astype(jnp.float32) + SMIN
    out_ref[...] = jnp.maximum(1.0 - jnp.abs(sup - tt[:, None, :]), 0.0)


def kernel(target_value, supports):
    xt = target_value.T
    grid = (NS + CB - 1) // CB
    out = pl.pallas_call(
        functools.partial(_tent_block, cb=CB),
        grid=(grid,),
        in_specs=[pl.BlockSpec((K, B), lambda j: (0, 0))],
        out_specs=pl.BlockSpec((K, CB, B), lambda j: (0, j, 0)),
        out_shape=jax.ShapeDtypeStruct((K, NS, B), jnp.float32),
        scratch_shapes=[pltpu.VMEM((K, B), jnp.float32)],
    )(xt)
    return jnp.transpose(out, (2, 0, 1))


# speedup vs baseline: 1.0520x; 1.0520x over previous
import functools

import jax
import jax.numpy as jnp
from jax import lax
from jax.experimental import pallas as pl
from jax.experimental.pallas import tpu as pltpu

EPS = 0.001
NS = 601
SMIN = -300.0

B, K = 4096, 50
CB = 16


def _tent_block(xt_ref, out_ref, tt_scr, *, cb):
    j = pl.program_id(0)

    @pl.when(j == 0)
    def _():
        x = xt_ref[...]
        tt_scr[...] = jnp.sign(x) * (
            jnp.sqrt(jnp.abs(x) + 1.0) - 1.0 + EPS * x)

    tt = tt_scr[...]
    col = lax.broadcasted_iota(jnp.int32, (K, cb, B), 1) + j * cb
    sup = col.astype(jnp.float32) + SMIN
    out_ref[...] = jnp.maximum(1.0 - jnp.abs(sup - tt[:, None, :]), 0.0)


def kernel(target_value, supports):
    xt = target_value.T
    grid = (NS + CB - 1) // CB
    out = pl.pallas_call(
        functools.partial(_tent_block, cb=CB),
        grid=(grid,),
        in_specs=[pl.BlockSpec((K, B), lambda j: (0, 0))],
        out_specs=pl.BlockSpec((K, CB, B), lambda j: (0, j, 0)),
        out_shape=jax.ShapeDtypeStruct((K, NS, B), jnp.float32),
        scratch_shapes=[pltpu.VMEM((K, B), jnp.float32)],
    )(xt)
    return jnp.transpose(out, (2, 0, 1))
